# baseline (speedup 1.0000x reference)
import functools

import jax
import jax.numpy as jnp
from jax import lax
from jax.experimental import pallas as pl
from jax.experimental.pallas import tpu as pltpu
from jax.experimental.pallas import tpu_sc as plsc

N_NODES = 10000
N_EDGES = 320000
N_GRAPHS = 100
D = 128
NC = 2
NS = 16
NW = NC * NS
K = 128
C0 = 88
C1 = 70
E_PAD = NS * (C0 + C1) * K
GARBAGE_ROW = N_NODES
N_ACC = 10240
RPT = N_ACC // NS

BR = 2000
GRID = N_NODES // BR


def _sc_agg_body(h_hbm, src_hbm, dst_hbm, zero_hbm, out_hbm,
                 sidx0, didx0, rows0, sidx1, didx1, rows1, acc, sem0, sem1):
    cid = lax.axis_index("c")
    tid = lax.axis_index("s")

    pltpu.sync_copy(zero_hbm, acc.at[pl.ds(tid * RPT, RPT)])
    plsc.subcore_barrier()

    nchunks = jnp.where(cid == 0, C0, C1)
    base = jnp.where(cid == 0, tid * C0, NS * C0 + tid * C1) * K

    bufs = ((sidx0, didx0, rows0, sem0), (sidx1, didx1, rows1, sem1))

    def stage(b, c):
        sidx, didx, rows, sem = bufs[b]
        off = base + c * K
        pltpu.sync_copy(src_hbm.at[pl.ds(off, K)], sidx)
        pltpu.sync_copy(dst_hbm.at[pl.ds(off, K)], didx)
        pltpu.async_copy(h_hbm.at[sidx], rows, sem)

    @pl.when(nchunks > 0)
    def _p0():
        stage(0, 0)

    @pl.when(nchunks > 1)
    def _p1():
        stage(1, 1)

    def body(i, carry):
        for b in range(2):
            c = 2 * i + b
            sidx, didx, rows, sem = bufs[b]
            pltpu.make_async_copy(h_hbm.at[sidx], rows, sem).wait()
            pltpu.sync_copy(rows, acc.at[didx], add=True)

            @pl.when(c + 2 < nchunks)
            def _prefetch(b=b, c=c):
                stage(b, c + 2)
        return carry

    lax.fori_loop(0, nchunks // 2, body, 0, unroll=False)
    plsc.subcore_barrier()

    pltpu.sync_copy(acc.at[pl.ds(tid * RPT, RPT)],
                    out_hbm.at[pl.ds(cid * N_ACC + tid * RPT, RPT)])


@functools.cache
def _sc_agg_kernel():
    return pl.kernel(
        _sc_agg_body,
        out_type=jax.ShapeDtypeStruct((NC * N_ACC, D), jnp.float32),
        mesh=plsc.VectorSubcoreMesh(core_axis_name="c", subcore_axis_name="s",
                                    num_cores=NC, num_subcores=NS),
        scratch_types=[
            pltpu.VMEM((K,), jnp.int32),
            pltpu.VMEM((K,), jnp.int32),
            pltpu.VMEM((K, D), jnp.float32),
            pltpu.VMEM((K,), jnp.int32),
            pltpu.VMEM((K,), jnp.int32),
            pltpu.VMEM((K, D), jnp.float32),
            pltpu.VMEM_SHARED((N_ACC, D), jnp.float32),
            pltpu.SemaphoreType.DMA,
            pltpu.SemaphoreType.DMA,
        ],
    )


def _tc_conv_body(h_ref, a0_ref, a1_ref, gid_ref, w1_ref, b1_ref, w2_ref,
                  b2_ref, hout_ref, psum_ref, cnt_ref):
    i = pl.program_id(0)
    z = h_ref[...] + a0_ref[0] + a1_ref[0]
    t = jnp.maximum(
        jnp.dot(z, w1_ref[...], preferred_element_type=jnp.float32,
                precision=lax.Precision.HIGHEST)
        + b1_ref[...], 0.0)
    h_new = jnp.maximum(
        jnp.dot(t, w2_ref[...], preferred_element_type=jnp.float32,
                precision=lax.Precision.HIGHEST)
        + b2_ref[...], 0.0)
    hout_ref[...] = h_new

    gids = gid_ref[0, 0]
    oh_t = (lax.broadcasted_iota(jnp.int32, (N_GRAPHS, BR), 0)
            == gids[None, :]).astype(jnp.float32)

    @pl.when(i == 0)
    def _():
        psum_ref[...] = jnp.zeros_like(psum_ref)
        cnt_ref[...] = jnp.zeros_like(cnt_ref)

    psum_ref[...] += jnp.dot(oh_t, h_new, preferred_element_type=jnp.float32,
                precision=lax.Precision.HIGHEST)
    cnt_ref[...] += jnp.sum(oh_t, axis=1, keepdims=True)


_tc_conv = pl.pallas_call(
    _tc_conv_body,
    grid=(GRID,),
    in_specs=[
        pl.BlockSpec((BR, D), lambda i: (i, 0)),
        pl.BlockSpec((1, BR, D), lambda i: (0, i, 0)),
        pl.BlockSpec((1, BR, D), lambda i: (1, i, 0)),
        pl.BlockSpec((1, 1, BR), lambda i: (i, 0, 0)),
        pl.BlockSpec((D, D), lambda i: (0, 0)),
        pl.BlockSpec((1, D), lambda i: (0, 0)),
        pl.BlockSpec((D, D), lambda i: (0, 0)),
        pl.BlockSpec((1, D), lambda i: (0, 0)),
    ],
    out_specs=[
        pl.BlockSpec((BR, D), lambda i: (i, 0)),
        pl.BlockSpec((N_GRAPHS, D), lambda i: (0, 0)),
        pl.BlockSpec((N_GRAPHS, 1), lambda i: (0, 0)),
    ],
    out_shape=[
        jax.ShapeDtypeStruct((N_NODES, D), jnp.float32),
        jax.ShapeDtypeStruct((N_GRAPHS, D), jnp.float32),
        jax.ShapeDtypeStruct((N_GRAPHS, 1), jnp.float32),
    ],
)


def _tc_head_body(p0_ref, p1_ref, cnt_ref, ev_ref, w0a_ref, w0b_ref, b0_ref,
                  *rest):
    out_ref = rest[-1]
    lin_refs = rest[:-1]
    cnt = jnp.maximum(cnt_ref[...], 1.0)
    hg = (p0_ref[...] + 2.0 * p1_ref[...]) / cnt
    x = jnp.dot(hg, w0a_ref[...], preferred_element_type=jnp.float32,
                precision=lax.Precision.HIGHEST)
    x += jnp.dot(ev_ref[...], w0b_ref[...], preferred_element_type=jnp.float32,
                precision=lax.Precision.HIGHEST)
    x = jnp.maximum(x + b0_ref[...], 0.0)
    n_lin = len(lin_refs) // 2
    for j in range(n_lin):
        w_ref, b_ref = lin_refs[2 * j], lin_refs[2 * j + 1]
        x = jnp.dot(x, w_ref[...], preferred_element_type=jnp.float32,
                precision=lax.Precision.HIGHEST) \
            + b_ref[...]
        if j < n_lin - 1:
            x = jnp.maximum(x, 0.0)
    out_ref[...] = x


def _head_call(p0, p1, cnt, ev, w0a, w0b, b0, lins):
    args = [p0, p1, cnt, ev, w0a, w0b, b0]
    for (w, b) in lins:
        args.append(w)
        args.append(b)
    return pl.pallas_call(
        _tc_head_body,
        out_shape=jax.ShapeDtypeStruct((N_GRAPHS, 1), jnp.float32),
    )(*args)


def kernel(in_feat, edge_index, graph_ids, event_feats,
           conv0_W1, conv0_b1, conv0_W2, conv0_b2,
           conv1_W1, conv1_b1, conv1_W2, conv1_b2,
           lin0_W, lin0_b, lin1_W, lin1_b, lin2_W, lin2_b, lin3_W, lin3_b,
           lin4_W, lin4_b, lin5_W, lin5_b, lin6_W, lin6_b):
    npad = E_PAD - N_EDGES
    src_p = jnp.concatenate([edge_index[0],
                             jnp.zeros((npad,), jnp.int32)])
    dst_p = jnp.concatenate([edge_index[1],
                             jnp.full((npad,), GARBAGE_ROW, jnp.int32)])
    zeros_blk = jnp.zeros((RPT, D), jnp.float32)
    gid3 = graph_ids.reshape(GRID, 1, BR)

    b01 = conv0_b1.reshape(1, D)
    b02 = conv0_b2.reshape(1, D)
    b11 = conv1_b1.reshape(1, D)
    b12 = conv1_b2.reshape(1, D)

    sc_agg = _sc_agg_kernel()
    a0 = sc_agg(in_feat, src_p, dst_p, zeros_blk).reshape(NC, N_ACC, D)
    h1, psum0, cnt = _tc_conv(in_feat, a0, a0, gid3,
                              conv0_W1, b01, conv0_W2, b02)
    a1 = sc_agg(h1, src_p, dst_p, zeros_blk).reshape(NC, N_ACC, D)
    h2, psum1, _ = _tc_conv(h1, a1, a1, gid3, conv1_W1, b11, conv1_W2, b12)

    lins = [(lin1_W, lin1_b.reshape(1, -1)), (lin2_W, lin2_b.reshape(1, -1)),
            (lin3_W, lin3_b.reshape(1, -1)), (lin4_W, lin4_b.reshape(1, -1)),
            (lin5_W, lin5_b.reshape(1, -1)), (lin6_W, lin6_b.reshape(1, -1))]
    return _head_call(psum0, psum1, cnt, event_feats,
                      lin0_W[:D], lin0_W[D:], lin0_b.reshape(1, -1), lins)


# speedup vs baseline: 1.1234x; 1.1234x over previous
import functools

import jax
import jax.numpy as jnp
from jax import lax
from jax.experimental import pallas as pl
from jax.experimental.pallas import tpu as pltpu
from jax.experimental.pallas import tpu_sc as plsc

N_NODES = 10000
N_EDGES = 320000
N_GRAPHS = 100
D = 128
NC = 2
NS = 16
NW = NC * NS
K = 128
C0 = 112
C1 = 46
E_PAD = NS * (C0 + C1) * K
GARBAGE_ROW = N_NODES
N_ACC = 10240
RPT = N_ACC // NS

BR = 2000
GRID = N_NODES // BR


def _sc_agg_body(h_hbm, src_hbm, dst_hbm, zero_hbm, out_hbm,
                 sidx0, didx0, rows0, sidx1, didx1, rows1, acc, sem0, sem1):
    cid = lax.axis_index("c")
    tid = lax.axis_index("s")

    pltpu.sync_copy(zero_hbm, acc.at[pl.ds(tid * RPT, RPT)])
    plsc.subcore_barrier()

    nchunks = jnp.where(cid == 0, C0, C1)
    base = jnp.where(cid == 0, tid * C0, NS * C0 + tid * C1) * K

    bufs = ((sidx0, didx0, rows0, sem0), (sidx1, didx1, rows1, sem1))

    def stage(b, c):
        sidx, didx, rows, sem = bufs[b]
        off = base + c * K
        pltpu.sync_copy(src_hbm.at[pl.ds(off, K)], sidx)
        pltpu.sync_copy(dst_hbm.at[pl.ds(off, K)], didx)
        pltpu.async_copy(h_hbm.at[sidx], rows, sem)

    @pl.when(nchunks > 0)
    def _p0():
        stage(0, 0)

    @pl.when(nchunks > 1)
    def _p1():
        stage(1, 1)

    def body(i, carry):
        for b in range(2):
            c = 2 * i + b
            sidx, didx, rows, sem = bufs[b]
            pltpu.make_async_copy(h_hbm.at[sidx], rows, sem).wait()
            pltpu.sync_copy(rows, acc.at[didx], add=True)

            @pl.when(c + 2 < nchunks)
            def _prefetch(b=b, c=c):
                stage(b, c + 2)
        return carry

    lax.fori_loop(0, nchunks // 2, body, 0, unroll=False)
    plsc.subcore_barrier()

    pltpu.sync_copy(acc.at[pl.ds(tid * RPT, RPT)],
                    out_hbm.at[pl.ds(cid * N_ACC + tid * RPT, RPT)])


@functools.cache
def _sc_agg_kernel():
    return pl.kernel(
        _sc_agg_body,
        out_type=jax.ShapeDtypeStruct((NC * N_ACC, D), jnp.float32),
        mesh=plsc.VectorSubcoreMesh(core_axis_name="c", subcore_axis_name="s",
                                    num_cores=NC, num_subcores=NS),
        scratch_types=[
            pltpu.VMEM((K,), jnp.int32),
            pltpu.VMEM((K,), jnp.int32),
            pltpu.VMEM((K, D), jnp.float32),
            pltpu.VMEM((K,), jnp.int32),
            pltpu.VMEM((K,), jnp.int32),
            pltpu.VMEM((K, D), jnp.float32),
            pltpu.VMEM_SHARED((N_ACC, D), jnp.float32),
            pltpu.SemaphoreType.DMA,
            pltpu.SemaphoreType.DMA,
        ],
    )


def _tc_conv_body(h_ref, a0_ref, a1_ref, gid_ref, w1_ref, b1_ref, w2_ref,
                  b2_ref, hout_ref, psum_ref, cnt_ref):
    i = pl.program_id(0)
    z = h_ref[...] + a0_ref[0] + a1_ref[0]
    t = jnp.maximum(
        jnp.dot(z, w1_ref[...], preferred_element_type=jnp.float32,
                precision=lax.Precision.HIGHEST)
        + b1_ref[...], 0.0)
    h_new = jnp.maximum(
        jnp.dot(t, w2_ref[...], preferred_element_type=jnp.float32,
                precision=lax.Precision.HIGHEST)
        + b2_ref[...], 0.0)
    hout_ref[...] = h_new

    gids = gid_ref[0, 0]
    oh_t = (lax.broadcasted_iota(jnp.int32, (N_GRAPHS, BR), 0)
            == gids[None, :]).astype(jnp.float32)

    @pl.when(i == 0)
    def _():
        psum_ref[...] = jnp.zeros_like(psum_ref)
        cnt_ref[...] = jnp.zeros_like(cnt_ref)

    psum_ref[...] += jnp.dot(oh_t, h_new, preferred_element_type=jnp.float32,
                precision=lax.Precision.HIGHEST)
    cnt_ref[...] += jnp.sum(oh_t, axis=1, keepdims=True)


_tc_conv = pl.pallas_call(
    _tc_conv_body,
    grid=(GRID,),
    in_specs=[
        pl.BlockSpec((BR, D), lambda i: (i, 0)),
        pl.BlockSpec((1, BR, D), lambda i: (0, i, 0)),
        pl.BlockSpec((1, BR, D), lambda i: (1, i, 0)),
        pl.BlockSpec((1, 1, BR), lambda i: (i, 0, 0)),
        pl.BlockSpec((D, D), lambda i: (0, 0)),
        pl.BlockSpec((1, D), lambda i: (0, 0)),
        pl.BlockSpec((D, D), lambda i: (0, 0)),
        pl.BlockSpec((1, D), lambda i: (0, 0)),
    ],
    out_specs=[
        pl.BlockSpec((BR, D), lambda i: (i, 0)),
        pl.BlockSpec((N_GRAPHS, D), lambda i: (0, 0)),
        pl.BlockSpec((N_GRAPHS, 1), lambda i: (0, 0)),
    ],
    out_shape=[
        jax.ShapeDtypeStruct((N_NODES, D), jnp.float32),
        jax.ShapeDtypeStruct((N_GRAPHS, D), jnp.float32),
        jax.ShapeDtypeStruct((N_GRAPHS, 1), jnp.float32),
    ],
)


def _tc_head_body(p0_ref, p1_ref, cnt_ref, ev_ref, w0a_ref, w0b_ref, b0_ref,
                  *rest):
    out_ref = rest[-1]
    lin_refs = rest[:-1]
    cnt = jnp.maximum(cnt_ref[...], 1.0)
    hg = (p0_ref[...] + 2.0 * p1_ref[...]) / cnt
    x = jnp.dot(hg, w0a_ref[...], preferred_element_type=jnp.float32,
                precision=lax.Precision.HIGHEST)
    x += jnp.dot(ev_ref[...], w0b_ref[...], preferred_element_type=jnp.float32,
                precision=lax.Precision.HIGHEST)
    x = jnp.maximum(x + b0_ref[...], 0.0)
    n_lin = len(lin_refs) // 2
    for j in range(n_lin):
        w_ref, b_ref = lin_refs[2 * j], lin_refs[2 * j + 1]
        x = jnp.dot(x, w_ref[...], preferred_element_type=jnp.float32,
                precision=lax.Precision.HIGHEST) \
            + b_ref[...]
        if j < n_lin - 1:
            x = jnp.maximum(x, 0.0)
    out_ref[...] = x


def _head_call(p0, p1, cnt, ev, w0a, w0b, b0, lins):
    args = [p0, p1, cnt, ev, w0a, w0b, b0]
    for (w, b) in lins:
        args.append(w)
        args.append(b)
    return pl.pallas_call(
        _tc_head_body,
        out_shape=jax.ShapeDtypeStruct((N_GRAPHS, 1), jnp.float32),
    )(*args)


def kernel(in_feat, edge_index, graph_ids, event_feats,
           conv0_W1, conv0_b1, conv0_W2, conv0_b2,
           conv1_W1, conv1_b1, conv1_W2, conv1_b2,
           lin0_W, lin0_b, lin1_W, lin1_b, lin2_W, lin2_b, lin3_W, lin3_b,
           lin4_W, lin4_b, lin5_W, lin5_b, lin6_W, lin6_b):
    npad = E_PAD - N_EDGES
    src_p = jnp.concatenate([edge_index[0],
                             jnp.zeros((npad,), jnp.int32)])
    dst_p = jnp.concatenate([edge_index[1],
                             jnp.full((npad,), GARBAGE_ROW, jnp.int32)])
    zeros_blk = jnp.zeros((RPT, D), jnp.float32)
    gid3 = graph_ids.reshape(GRID, 1, BR)

    b01 = conv0_b1.reshape(1, D)
    b02 = conv0_b2.reshape(1, D)
    b11 = conv1_b1.reshape(1, D)
    b12 = conv1_b2.reshape(1, D)

    sc_agg = _sc_agg_kernel()
    a0 = sc_agg(in_feat, src_p, dst_p, zeros_blk).reshape(NC, N_ACC, D)
    h1, psum0, cnt = _tc_conv(in_feat, a0, a0, gid3,
                              conv0_W1, b01, conv0_W2, b02)
    a1 = sc_agg(h1, src_p, dst_p, zeros_blk).reshape(NC, N_ACC, D)
    h2, psum1, _ = _tc_conv(h1, a1, a1, gid3, conv1_W1, b11, conv1_W2, b12)

    lins = [(lin1_W, lin1_b.reshape(1, -1)), (lin2_W, lin2_b.reshape(1, -1)),
            (lin3_W, lin3_b.reshape(1, -1)), (lin4_W, lin4_b.reshape(1, -1)),
            (lin5_W, lin5_b.reshape(1, -1)), (lin6_W, lin6_b.reshape(1, -1))]
    return _head_call(psum0, psum1, cnt, event_feats,
                      lin0_W[:D], lin0_W[D:], lin0_b.reshape(1, -1), lins)


# speedup vs baseline: 1.1849x; 1.0547x over previous
import functools

import jax
import jax.numpy as jnp
from jax import lax
from jax.experimental import pallas as pl
from jax.experimental.pallas import tpu as pltpu
from jax.experimental.pallas import tpu_sc as plsc

N_NODES = 10000
N_EDGES = 320000
N_GRAPHS = 100
D = 128
NC = 2
NS = 16
NW = NC * NS
K = 128
C0 = 122
C1 = 36
E_PAD = NS * (C0 + C1) * K
GARBAGE_ROW = N_NODES
N_ACC = 10240
RPT = N_ACC // NS

BR = 2000
GRID = N_NODES // BR


def _sc_agg_body(h_hbm, src_hbm, dst_hbm, zero_hbm, out_hbm,
                 sidx0, didx0, rows0, sidx1, didx1, rows1, acc, sem0, sem1):
    cid = lax.axis_index("c")
    tid = lax.axis_index("s")

    pltpu.sync_copy(zero_hbm, acc.at[pl.ds(tid * RPT, RPT)])
    plsc.subcore_barrier()

    nchunks = jnp.where(cid == 0, C0, C1)
    base = jnp.where(cid == 0, tid * C0, NS * C0 + tid * C1) * K

    bufs = ((sidx0, didx0, rows0, sem0), (sidx1, didx1, rows1, sem1))

    def stage(b, c):
        sidx, didx, rows, sem = bufs[b]
        off = base + c * K
        pltpu.sync_copy(src_hbm.at[pl.ds(off, K)], sidx)
        pltpu.sync_copy(dst_hbm.at[pl.ds(off, K)], didx)
        pltpu.async_copy(h_hbm.at[sidx], rows, sem)

    @pl.when(nchunks > 0)
    def _p0():
        stage(0, 0)

    @pl.when(nchunks > 1)
    def _p1():
        stage(1, 1)

    def body(i, carry):
        for b in range(2):
            c = 2 * i + b
            sidx, didx, rows, sem = bufs[b]
            pltpu.make_async_copy(h_hbm.at[sidx], rows, sem).wait()
            pltpu.sync_copy(rows, acc.at[didx], add=True)

            @pl.when(c + 2 < nchunks)
            def _prefetch(b=b, c=c):
                stage(b, c + 2)
        return carry

    lax.fori_loop(0, nchunks // 2, body, 0, unroll=False)
    plsc.subcore_barrier()

    pltpu.sync_copy(acc.at[pl.ds(tid * RPT, RPT)],
                    out_hbm.at[pl.ds(cid * N_ACC + tid * RPT, RPT)])


@functools.cache
def _sc_agg_kernel():
    return pl.kernel(
        _sc_agg_body,
        out_type=jax.ShapeDtypeStruct((NC * N_ACC, D), jnp.float32),
        mesh=plsc.VectorSubcoreMesh(core_axis_name="c", subcore_axis_name="s",
                                    num_cores=NC, num_subcores=NS),
        scratch_types=[
            pltpu.VMEM((K,), jnp.int32),
            pltpu.VMEM((K,), jnp.int32),
            pltpu.VMEM((K, D), jnp.float32),
            pltpu.VMEM((K,), jnp.int32),
            pltpu.VMEM((K,), jnp.int32),
            pltpu.VMEM((K, D), jnp.float32),
            pltpu.VMEM_SHARED((N_ACC, D), jnp.float32),
            pltpu.SemaphoreType.DMA,
            pltpu.SemaphoreType.DMA,
        ],
    )


def _tc_conv_body(h_ref, a0_ref, a1_ref, gid_ref, w1_ref, b1_ref, w2_ref,
                  b2_ref, hout_ref, psum_ref, cnt_ref):
    i = pl.program_id(0)
    z = h_ref[...] + a0_ref[0] + a1_ref[0]
    t = jnp.maximum(
        jnp.dot(z, w1_ref[...], preferred_element_type=jnp.float32,
                precision=lax.Precision.HIGHEST)
        + b1_ref[...], 0.0)
    h_new = jnp.maximum(
        jnp.dot(t, w2_ref[...], preferred_element_type=jnp.float32,
                precision=lax.Precision.HIGHEST)
        + b2_ref[...], 0.0)
    hout_ref[...] = h_new

    gids = gid_ref[0, 0]
    oh_t = (lax.broadcasted_iota(jnp.int32, (N_GRAPHS, BR), 0)
            == gids[None, :]).astype(jnp.float32)

    @pl.when(i == 0)
    def _():
        psum_ref[...] = jnp.zeros_like(psum_ref)
        cnt_ref[...] = jnp.zeros_like(cnt_ref)

    psum_ref[...] += jnp.dot(oh_t, h_new, preferred_element_type=jnp.float32,
                precision=lax.Precision.HIGHEST)
    cnt_ref[...] += jnp.sum(oh_t, axis=1, keepdims=True)


_tc_conv = pl.pallas_call(
    _tc_conv_body,
    grid=(GRID,),
    in_specs=[
        pl.BlockSpec((BR, D), lambda i: (i, 0)),
        pl.BlockSpec((1, BR, D), lambda i: (0, i, 0)),
        pl.BlockSpec((1, BR, D), lambda i: (1, i, 0)),
        pl.BlockSpec((1, 1, BR), lambda i: (i, 0, 0)),
        pl.BlockSpec((D, D), lambda i: (0, 0)),
        pl.BlockSpec((1, D), lambda i: (0, 0)),
        pl.BlockSpec((D, D), lambda i: (0, 0)),
        pl.BlockSpec((1, D), lambda i: (0, 0)),
    ],
    out_specs=[
        pl.BlockSpec((BR, D), lambda i: (i, 0)),
        pl.BlockSpec((N_GRAPHS, D), lambda i: (0, 0)),
        pl.BlockSpec((N_GRAPHS, 1), lambda i: (0, 0)),
    ],
    out_shape=[
        jax.ShapeDtypeStruct((N_NODES, D), jnp.float32),
        jax.ShapeDtypeStruct((N_GRAPHS, D), jnp.float32),
        jax.ShapeDtypeStruct((N_GRAPHS, 1), jnp.float32),
    ],
)


def _tc_head_body(p0_ref, p1_ref, cnt_ref, ev_ref, w0a_ref, w0b_ref, b0_ref,
                  *rest):
    out_ref = rest[-1]
    lin_refs = rest[:-1]
    cnt = jnp.maximum(cnt_ref[...], 1.0)
    hg = (p0_ref[...] + 2.0 * p1_ref[...]) / cnt
    x = jnp.dot(hg, w0a_ref[...], preferred_element_type=jnp.float32,
                precision=lax.Precision.HIGHEST)
    x += jnp.dot(ev_ref[...], w0b_ref[...], preferred_element_type=jnp.float32,
                precision=lax.Precision.HIGHEST)
    x = jnp.maximum(x + b0_ref[...], 0.0)
    n_lin = len(lin_refs) // 2
    for j in range(n_lin):
        w_ref, b_ref = lin_refs[2 * j], lin_refs[2 * j + 1]
        x = jnp.dot(x, w_ref[...], preferred_element_type=jnp.float32,
                precision=lax.Precision.HIGHEST) \
            + b_ref[...]
        if j < n_lin - 1:
            x = jnp.maximum(x, 0.0)
    out_ref[...] = x


def _head_call(p0, p1, cnt, ev, w0a, w0b, b0, lins):
    args = [p0, p1, cnt, ev, w0a, w0b, b0]
    for (w, b) in lins:
        args.append(w)
        args.append(b)
    return pl.pallas_call(
        _tc_head_body,
        out_shape=jax.ShapeDtypeStruct((N_GRAPHS, 1), jnp.float32),
    )(*args)


def kernel(in_feat, edge_index, graph_ids, event_feats,
           conv0_W1, conv0_b1, conv0_W2, conv0_b2,
           conv1_W1, conv1_b1, conv1_W2, conv1_b2,
           lin0_W, lin0_b, lin1_W, lin1_b, lin2_W, lin2_b, lin3_W, lin3_b,
           lin4_W, lin4_b, lin5_W, lin5_b, lin6_W, lin6_b):
    npad = E_PAD - N_EDGES
    src_p = jnp.concatenate([edge_index[0],
                             jnp.zeros((npad,), jnp.int32)])
    dst_p = jnp.concatenate([edge_index[1],
                             jnp.full((npad,), GARBAGE_ROW, jnp.int32)])
    zeros_blk = jnp.zeros((RPT, D), jnp.float32)
    gid3 = graph_ids.reshape(GRID, 1, BR)

    b01 = conv0_b1.reshape(1, D)
    b02 = conv0_b2.reshape(1, D)
    b11 = conv1_b1.reshape(1, D)
    b12 = conv1_b2.reshape(1, D)

    sc_agg = _sc_agg_kernel()
    a0 = sc_agg(in_feat, src_p, dst_p, zeros_blk).reshape(NC, N_ACC, D)
    h1, psum0, cnt = _tc_conv(in_feat, a0, a0, gid3,
                              conv0_W1, b01, conv0_W2, b02)
    a1 = sc_agg(h1, src_p, dst_p, zeros_blk).reshape(NC, N_ACC, D)
    h2, psum1, _ = _tc_conv(h1, a1, a1, gid3, conv1_W1, b11, conv1_W2, b12)

    lins = [(lin1_W, lin1_b.reshape(1, -1)), (lin2_W, lin2_b.reshape(1, -1)),
            (lin3_W, lin3_b.reshape(1, -1)), (lin4_W, lin4_b.reshape(1, -1)),
            (lin5_W, lin5_b.reshape(1, -1)), (lin6_W, lin6_b.reshape(1, -1))]
    return _head_call(psum0, psum1, cnt, event_feats,
                      lin0_W[:D], lin0_W[D:], lin0_b.reshape(1, -1), lins)


# speedup vs baseline: 1.1890x; 1.0035x over previous
import functools

import jax
import jax.numpy as jnp
from jax import lax
from jax.experimental import pallas as pl
from jax.experimental.pallas import tpu as pltpu
from jax.experimental.pallas import tpu_sc as plsc

N_NODES = 10000
N_EDGES = 320000
N_GRAPHS = 100
D = 128
NC = 2
NS = 16
NW = NC * NS
K = 128
C0 = 132
C1 = 26
E_PAD = NS * (C0 + C1) * K
GARBAGE_ROW = N_NODES
N_ACC = 10240
RPT = N_ACC // NS

BR = 2000
GRID = N_NODES // BR


def _sc_agg_body(h_hbm, src_hbm, dst_hbm, zero_hbm, out_hbm,
                 sidx0, didx0, rows0, sidx1, didx1, rows1, acc, sem0, sem1):
    cid = lax.axis_index("c")
    tid = lax.axis_index("s")

    pltpu.sync_copy(zero_hbm, acc.at[pl.ds(tid * RPT, RPT)])
    plsc.subcore_barrier()

    nchunks = jnp.where(cid == 0, C0, C1)
    base = jnp.where(cid == 0, tid * C0, NS * C0 + tid * C1) * K

    bufs = ((sidx0, didx0, rows0, sem0), (sidx1, didx1, rows1, sem1))

    def stage(b, c):
        sidx, didx, rows, sem = bufs[b]
        off = base + c * K
        pltpu.sync_copy(src_hbm.at[pl.ds(off, K)], sidx)
        pltpu.sync_copy(dst_hbm.at[pl.ds(off, K)], didx)
        pltpu.async_copy(h_hbm.at[sidx], rows, sem)

    @pl.when(nchunks > 0)
    def _p0():
        stage(0, 0)

    @pl.when(nchunks > 1)
    def _p1():
        stage(1, 1)

    def body(i, carry):
        for b in range(2):
            c = 2 * i + b
            sidx, didx, rows, sem = bufs[b]
            pltpu.make_async_copy(h_hbm.at[sidx], rows, sem).wait()
            pltpu.sync_copy(rows, acc.at[didx], add=True)

            @pl.when(c + 2 < nchunks)
            def _prefetch(b=b, c=c):
                stage(b, c + 2)
        return carry

    lax.fori_loop(0, nchunks // 2, body, 0, unroll=False)
    plsc.subcore_barrier()

    pltpu.sync_copy(acc.at[pl.ds(tid * RPT, RPT)],
                    out_hbm.at[pl.ds(cid * N_ACC + tid * RPT, RPT)])


@functools.cache
def _sc_agg_kernel():
    return pl.kernel(
        _sc_agg_body,
        out_type=jax.ShapeDtypeStruct((NC * N_ACC, D), jnp.float32),
        mesh=plsc.VectorSubcoreMesh(core_axis_name="c", subcore_axis_name="s",
                                    num_cores=NC, num_subcores=NS),
        scratch_types=[
            pltpu.VMEM((K,), jnp.int32),
            pltpu.VMEM((K,), jnp.int32),
            pltpu.VMEM((K, D), jnp.float32),
            pltpu.VMEM((K,), jnp.int32),
            pltpu.VMEM((K,), jnp.int32),
            pltpu.VMEM((K, D), jnp.float32),
            pltpu.VMEM_SHARED((N_ACC, D), jnp.float32),
            pltpu.SemaphoreType.DMA,
            pltpu.SemaphoreType.DMA,
        ],
    )


def _tc_conv_body(h_ref, a0_ref, a1_ref, gid_ref, w1_ref, b1_ref, w2_ref,
                  b2_ref, hout_ref, psum_ref, cnt_ref):
    i = pl.program_id(0)
    z = h_ref[...] + a0_ref[0] + a1_ref[0]
    t = jnp.maximum(
        jnp.dot(z, w1_ref[...], preferred_element_type=jnp.float32,
                precision=lax.Precision.HIGHEST)
        + b1_ref[...], 0.0)
    h_new = jnp.maximum(
        jnp.dot(t, w2_ref[...], preferred_element_type=jnp.float32,
                precision=lax.Precision.HIGHEST)
        + b2_ref[...], 0.0)
    hout_ref[...] = h_new

    gids = gid_ref[0, 0]
    oh_t = (lax.broadcasted_iota(jnp.int32, (N_GRAPHS, BR), 0)
            == gids[None, :]).astype(jnp.float32)

    @pl.when(i == 0)
    def _():
        psum_ref[...] = jnp.zeros_like(psum_ref)
        cnt_ref[...] = jnp.zeros_like(cnt_ref)

    psum_ref[...] += jnp.dot(oh_t, h_new, preferred_element_type=jnp.float32,
                precision=lax.Precision.HIGHEST)
    cnt_ref[...] += jnp.sum(oh_t, axis=1, keepdims=True)


_tc_conv = pl.pallas_call(
    _tc_conv_body,
    grid=(GRID,),
    in_specs=[
        pl.BlockSpec((BR, D), lambda i: (i, 0)),
        pl.BlockSpec((1, BR, D), lambda i: (0, i, 0)),
        pl.BlockSpec((1, BR, D), lambda i: (1, i, 0)),
        pl.BlockSpec((1, 1, BR), lambda i: (i, 0, 0)),
        pl.BlockSpec((D, D), lambda i: (0, 0)),
        pl.BlockSpec((1, D), lambda i: (0, 0)),
        pl.BlockSpec((D, D), lambda i: (0, 0)),
        pl.BlockSpec((1, D), lambda i: (0, 0)),
    ],
    out_specs=[
        pl.BlockSpec((BR, D), lambda i: (i, 0)),
        pl.BlockSpec((N_GRAPHS, D), lambda i: (0, 0)),
        pl.BlockSpec((N_GRAPHS, 1), lambda i: (0, 0)),
    ],
    out_shape=[
        jax.ShapeDtypeStruct((N_NODES, D), jnp.float32),
        jax.ShapeDtypeStruct((N_GRAPHS, D), jnp.float32),
        jax.ShapeDtypeStruct((N_GRAPHS, 1), jnp.float32),
    ],
)


def _tc_head_body(p0_ref, p1_ref, cnt_ref, ev_ref, w0a_ref, w0b_ref, b0_ref,
                  *rest):
    out_ref = rest[-1]
    lin_refs = rest[:-1]
    cnt = jnp.maximum(cnt_ref[...], 1.0)
    hg = (p0_ref[...] + 2.0 * p1_ref[...]) / cnt
    x = jnp.dot(hg, w0a_ref[...], preferred_element_type=jnp.float32,
                precision=lax.Precision.HIGHEST)
    x += jnp.dot(ev_ref[...], w0b_ref[...], preferred_element_type=jnp.float32,
                precision=lax.Precision.HIGHEST)
    x = jnp.maximum(x + b0_ref[...], 0.0)
    n_lin = len(lin_refs) // 2
    for j in range(n_lin):
        w_ref, b_ref = lin_refs[2 * j], lin_refs[2 * j + 1]
        x = jnp.dot(x, w_ref[...], preferred_element_type=jnp.float32,
                precision=lax.Precision.HIGHEST) \
            + b_ref[...]
        if j < n_lin - 1:
            x = jnp.maximum(x, 0.0)
    out_ref[...] = x


def _head_call(p0, p1, cnt, ev, w0a, w0b, b0, lins):
    args = [p0, p1, cnt, ev, w0a, w0b, b0]
    for (w, b) in lins:
        args.append(w)
        args.append(b)
    return pl.pallas_call(
        _tc_head_body,
        out_shape=jax.ShapeDtypeStruct((N_GRAPHS, 1), jnp.float32),
    )(*args)


def kernel(in_feat, edge_index, graph_ids, event_feats,
           conv0_W1, conv0_b1, conv0_W2, conv0_b2,
           conv1_W1, conv1_b1, conv1_W2, conv1_b2,
           lin0_W, lin0_b, lin1_W, lin1_b, lin2_W, lin2_b, lin3_W, lin3_b,
           lin4_W, lin4_b, lin5_W, lin5_b, lin6_W, lin6_b):
    npad = E_PAD - N_EDGES
    src_p = jnp.concatenate([edge_index[0],
                             jnp.zeros((npad,), jnp.int32)])
    dst_p = jnp.concatenate([edge_index[1],
                             jnp.full((npad,), GARBAGE_ROW, jnp.int32)])
    zeros_blk = jnp.zeros((RPT, D), jnp.float32)
    gid3 = graph_ids.reshape(GRID, 1, BR)

    b01 = conv0_b1.reshape(1, D)
    b02 = conv0_b2.reshape(1, D)
    b11 = conv1_b1.reshape(1, D)
    b12 = conv1_b2.reshape(1, D)

    sc_agg = _sc_agg_kernel()
    a0 = sc_agg(in_feat, src_p, dst_p, zeros_blk).reshape(NC, N_ACC, D)
    h1, psum0, cnt = _tc_conv(in_feat, a0, a0, gid3,
                              conv0_W1, b01, conv0_W2, b02)
    a1 = sc_agg(h1, src_p, dst_p, zeros_blk).reshape(NC, N_ACC, D)
    h2, psum1, _ = _tc_conv(h1, a1, a1, gid3, conv1_W1, b11, conv1_W2, b12)

    lins = [(lin1_W, lin1_b.reshape(1, -1)), (lin2_W, lin2_b.reshape(1, -1)),
            (lin3_W, lin3_b.reshape(1, -1)), (lin4_W, lin4_b.reshape(1, -1)),
            (lin5_W, lin5_b.reshape(1, -1)), (lin6_W, lin6_b.reshape(1, -1))]
    return _head_call(psum0, psum1, cnt, event_feats,
                      lin0_W[:D], lin0_W[D:], lin0_b.reshape(1, -1), lins)
